# baseline (speedup 1.0000x reference)
import jax
import jax.numpy as jnp
from jax import lax
from jax.experimental import pallas as pl
from jax.experimental.pallas import tpu as pltpu
from jax.experimental.pallas import tpu_sc as plsc

N = 10000
E = 320000
F = 128
FH = 32
FS = 4
H = 4
NCLASS = 64
NP = 10240
B = 128
NBLK = 157
EPT = NBLK * B
NTILE = 16
EPAD = NTILE * EPT
STRIPE = NP // NTILE
RB = 640
NEG = -1e30


def _proj_body(x_ref, w0_ref, w1_ref, al0_ref, ar0_ref, al1_ref, ar1_ref,
               ff_ref, el_ref, er_ref):
  xb = x_ref[...]
  for l in range(2):
    w_ref = (w0_ref, w1_ref)[l]
    al_ref = (al0_ref, al1_ref)[l]
    ar_ref = (ar0_ref, ar1_ref)[l]
    for h in range(H):
      f = jnp.dot(xb, w_ref[:, h * F:(h + 1) * F],
                  preferred_element_type=jnp.float32)
      for fh in range(FS):
        ff_ref[FS * (l * H + h) + fh] = f[:, fh * FH:(fh + 1) * FH]
      el_ref[l * H + h, :] = jnp.sum(f * al_ref[h, :][None, :], axis=-1)
      er_ref[l * H + h, :] = jnp.sum(f * ar_ref[h, :][None, :], axis=-1)


def _project(xp, W0, W1, al0, ar0, al1, ar1):
  full2 = lambda i: (0, 0)
  return pl.pallas_call(
      _proj_body,
      grid=(NP // RB,),
      in_specs=[
          pl.BlockSpec((RB, F), lambda i: (i, 0)),
          pl.BlockSpec((F, H * F), full2),
          pl.BlockSpec((F, H * F), full2),
          pl.BlockSpec((H, F), full2),
          pl.BlockSpec((H, F), full2),
          pl.BlockSpec((H, F), full2),
          pl.BlockSpec((H, F), full2),
      ],
      out_specs=[
          pl.BlockSpec((2 * H * FS, RB, FH), lambda i: (0, i, 0)),
          pl.BlockSpec((2 * H, RB), lambda i: (0, i)),
          pl.BlockSpec((2 * H, RB), lambda i: (0, i)),
      ],
      out_shape=[
          jax.ShapeDtypeStruct((2 * H * FS, NP, FH), jnp.float32),
          jax.ShapeDtypeStruct((2 * H, NP), jnp.float32),
          jax.ShapeDtypeStruct((2 * H, NP), jnp.float32),
      ],
  )(xp, W0, W1, al0, ar0, al1, ar1)


def _sc_body(srcs, dsts, ff, elcat, ercat,
             agg,
             src_all, dst_all, tab_a, tab_b, tab_c, abuf,
             in0, in1, out0, out1, bidx0, bidx1, sb0, sb1,
             den0, den1, acc,
             g0, g1, s0, s1, a0, a1):
  c = lax.axis_index("c")
  s = lax.axis_index("s")
  pltpu.sync_copy(srcs.at[s], src_all)
  pltpu.sync_copy(dsts.at[s], dst_all)

  zero16 = jnp.zeros((16,), jnp.float32)
  ins = (in0, in1)
  outs = (out0, out1)
  bidxs = (bidx0, bidx1)
  gsems = (g0, g1)
  ssems = (s0, s1)
  sbufs = (sb0, sb1)
  asems = (a0, a1)

  def _sval(j, k):
    sv = src_all[j, pl.ds(k * 16, 16)]
    dv = dst_all[j, pl.ds(k * 16, 16)]
    e = plsc.load_gather(tab_a, [sv]) + plsc.load_gather(tab_b, [dv])
    e = jnp.where(e > 0, e, 0.2 * e)
    return sv, dv, jnp.exp(e)

  def layer_body(li, carry):
    def _zt(i, carry1):
      tab_c[pl.ds(i * 16, 16)] = zero16
      return carry1
    lax.fori_loop(0, STRIPE // 16, _zt, 0)
    pltpu.sync_copy(tab_c.at[pl.ds(0, STRIPE)],
                    den0.at[pl.ds(s * STRIPE, STRIPE)])
    pltpu.sync_copy(tab_c.at[pl.ds(0, STRIPE)],
                    den1.at[pl.ds(s * STRIPE, STRIPE)])
    plsc.subcore_barrier()
    for hh in range(2):
      den = (den0, den1)[hh]
      lane = li * H + 2 * c + hh
      pltpu.sync_copy(elcat.at[pl.ds(lane * NP, NP)], tab_a)
      pltpu.sync_copy(ercat.at[pl.ds(lane * NP, NP)], tab_b)

      def stage1(j, b, do_wait):
        sb = sbufs[b]
        if do_wait:
          pltpu.make_async_copy(sb, den.at[dst_all.at[0]], asems[b]).wait()
        for k in range(8):
          _, _, sval = _sval(j, k)
          sb[pl.ds(k * 16, 16)] = sval
        pltpu.async_copy(sb, den.at[dst_all.at[j]], asems[b], add=True)

      stage1(0, 0, False)
      stage1(1, 1, False)

      def p1(g, carry1):
        stage1(2 * g, 0, True)
        stage1(2 * g + 1, 1, True)
        return carry1
      lax.fori_loop(1, NBLK // 2, p1, 0)
      stage1(NBLK - 1, 0, True)
      pltpu.make_async_copy(sb0, den.at[dst_all.at[0]], a0).wait()
      pltpu.make_async_copy(sb1, den.at[dst_all.at[0]], a1).wait()
    plsc.subcore_barrier()

    for hh in range(2):
      den = (den0, den1)[hh]
      lane = li * H + 2 * c + hh
      pltpu.sync_copy(elcat.at[pl.ds(lane * NP, NP)], tab_a)
      pltpu.sync_copy(ercat.at[pl.ds(lane * NP, NP)], tab_b)
      pltpu.sync_copy(den, tab_c)

      def ablk(j, carry1):
        for k in range(8):
          _, dv, sval = _sval(j, k)
          dval = plsc.load_gather(tab_c, [dv])
          abuf[pl.ds(j * B + k * 16, 16)] = sval / (dval + 1e-16)
        return carry1
      lax.fori_loop(0, NBLK, ablk, 0)

      def half_body(fh, carry1):
        base = (FS * lane + fh) * NP
        bvec = jnp.full((16,), base, jnp.int32)

        def _zo(r, carry2):
          for k in range(FH // 16):
            out0[r, pl.ds(k * 16, 16)] = zero16
          return carry2
        lax.fori_loop(0, B, _zo, 0)
        for q in range(STRIPE // B):
          pltpu.sync_copy(out0, acc.at[pl.ds(s * STRIPE + q * B, B)])
        plsc.subcore_barrier()

        def _bidx(j, b):
          for k in range(8):
            bidxs[b][pl.ds(k * 16, 16)] = (
                src_all[j, pl.ds(k * 16, 16)] + bvec)

        def _scale_static(j, b):
          jb = j * B
          for r in range(B):
            aspl = plsc.load_gather(
                abuf, [jnp.full((16,), jb + r, jnp.int32)])
            for k in range(FH // 16):
              outs[b][r, pl.ds(k * 16, 16)] = (
                  ins[b][r, pl.ds(k * 16, 16)] * aspl)

        def _scale_dyn(j, b):
          jb = j * B
          def rb(r, carry2):
            aspl = plsc.load_gather(
                abuf, [jnp.full((16,), jb + r, jnp.int32)])
            for k in range(FH // 16):
              outs[b][r, pl.ds(k * 16, 16)] = (
                  ins[b][r, pl.ds(k * 16, 16)] * aspl)
            return carry2
          lax.fori_loop(0, B, rb, 0)

        def stage2(j, b, do_swait, gnext, static_scale):
          pltpu.make_async_copy(ff.at[bidxs[b]], ins[b], gsems[b]).wait()
          if do_swait:
            pltpu.make_async_copy(
                outs[b], acc.at[dst_all.at[0]], ssems[b]).wait()
          if static_scale:
            _scale_static(j, b)
          else:
            _scale_dyn(j, b)
          pltpu.async_copy(outs[b], acc.at[dst_all.at[j]], ssems[b],
                           add=True)
          if gnext:
            @pl.when(j + 2 < NBLK)
            def _():
              _bidx(j + 2, b)
              pltpu.async_copy(ff.at[bidxs[b]], ins[b], gsems[b])

        _bidx(0, 0)
        pltpu.async_copy(ff.at[bidx0], in0, g0)
        _bidx(1, 1)
        pltpu.async_copy(ff.at[bidx1], in1, g1)
        stage2(0, 0, False, True, False)
        stage2(1, 1, False, True, False)

        def p2(g, carry2):
          stage2(2 * g, 0, True, True, True)
          stage2(2 * g + 1, 1, True, True, True)
          return carry2
        lax.fori_loop(1, NBLK // 2, p2, 0)
        stage2(NBLK - 1, 0, True, False, False)
        pltpu.make_async_copy(out0, acc.at[dst_all.at[0]], s0).wait()
        pltpu.make_async_copy(out1, acc.at[dst_all.at[0]], s1).wait()

        plsc.subcore_barrier()
        row0 = base + s * STRIPE
        pltpu.sync_copy(acc.at[pl.ds(s * STRIPE, STRIPE)],
                        agg.at[pl.ds(row0, STRIPE)])
        plsc.subcore_barrier()
        return carry1
      lax.fori_loop(0, FS, half_body, 0)
    return carry
  lax.fori_loop(0, 2, layer_body, 0)


def _sc_agg(srcs, dsts, ff, elcat, ercat):
  mesh = plsc.VectorSubcoreMesh(core_axis_name="c", subcore_axis_name="s")
  kfn = pl.kernel(
      _sc_body,
      out_type=jax.ShapeDtypeStruct((2 * H * FS * NP, FH), jnp.float32),
      mesh=mesh,
      compiler_params=pltpu.CompilerParams(needs_layout_passes=False,
                                           use_tc_tiling_on_sc=False),
      scratch_types=[
          pltpu.VMEM((NBLK, B), jnp.int32),
          pltpu.VMEM((NBLK, B), jnp.int32),
          pltpu.VMEM((NP,), jnp.float32),
          pltpu.VMEM((NP,), jnp.float32),
          pltpu.VMEM((NP,), jnp.float32),
          pltpu.VMEM((NBLK * B,), jnp.float32),
          pltpu.VMEM((B, FH), jnp.float32),
          pltpu.VMEM((B, FH), jnp.float32),
          pltpu.VMEM((B, FH), jnp.float32),
          pltpu.VMEM((B, FH), jnp.float32),
          pltpu.VMEM((B,), jnp.int32),
          pltpu.VMEM((B,), jnp.int32),
          pltpu.VMEM((B,), jnp.float32),
          pltpu.VMEM((B,), jnp.float32),
          pltpu.VMEM_SHARED((NP,), jnp.float32),
          pltpu.VMEM_SHARED((NP,), jnp.float32),
          pltpu.VMEM_SHARED((NP, FH), jnp.float32),
          pltpu.SemaphoreType.DMA,
          pltpu.SemaphoreType.DMA,
          pltpu.SemaphoreType.DMA,
          pltpu.SemaphoreType.DMA,
          pltpu.SemaphoreType.DMA,
          pltpu.SemaphoreType.DMA,
      ],
  )
  return kfn(srcs, dsts, ff, elcat, ercat)


def _out_body(agg_ref, wout_ref, bout_ref, o_ref):
  a = agg_ref[...]
  mls = []
  for l in range(2):
    parts = []
    for fh in range(FS):
      acc_fh = a[FS * (l * H + 0) + fh]
      for h in range(1, H):
        acc_fh = acc_fh + a[FS * (l * H + h) + fh]
      parts.append(acc_fh * 0.25)
    mls.append(jnp.concatenate(parts, axis=-1))
  h0 = jnp.maximum(mls[0], 0.0)
  h1 = jnp.maximum(h0 + mls[1], 0.0)
  lo = jnp.dot(h1, wout_ref[...], preferred_element_type=jnp.float32)
  lo = lo + bout_ref[...]
  z = lo - jnp.max(lo, axis=-1, keepdims=True)
  o_ref[...] = z - jnp.log(jnp.sum(jnp.exp(z), axis=-1, keepdims=True))


def _readout(agg, Wout, bout2):
  return pl.pallas_call(
      _out_body,
      grid=(NP // RB,),
      in_specs=[
          pl.BlockSpec((2 * H * FS, RB, FH), lambda i: (0, i, 0)),
          pl.BlockSpec((F, NCLASS), lambda i: (0, 0)),
          pl.BlockSpec((1, NCLASS), lambda i: (0, 0)),
      ],
      out_specs=pl.BlockSpec((RB, NCLASS), lambda i: (i, 0)),
      out_shape=jax.ShapeDtypeStruct((NP, NCLASS), jnp.float32),
  )(agg, Wout, bout2)


def kernel(x, edge_index, W0, al0, ar0, W1, al1, ar1, Wout, bout):
  src = edge_index[0].astype(jnp.int32)
  dst = edge_index[1].astype(jnp.int32)
  pad = EPAD - E
  srcs = jnp.concatenate([src, jnp.zeros((pad,), jnp.int32)])
  srcs = srcs.reshape(NTILE, NBLK, B)
  dsts = jnp.concatenate([dst, jnp.full((pad,), N, jnp.int32)])
  dsts = dsts.reshape(NTILE, NBLK, B)
  xp = jnp.pad(x, ((0, NP - N), (0, 0)))

  ff, el, er = _project(xp, W0, W1, al0, ar0, al1, ar1)
  er = er.at[:, N:].set(NEG)

  agg = _sc_agg(srcs, dsts, ff.reshape(2 * H * FS * NP, FH),
                el.reshape(2 * H * NP), er.reshape(2 * H * NP))

  out = _readout(agg.reshape(2 * H * FS, NP, FH), Wout,
                 bout.reshape(1, NCLASS).astype(jnp.float32))
  return out[:N]


# speedup vs baseline: 1.1119x; 1.1119x over previous
import jax
import jax.numpy as jnp
from jax import lax
from jax.experimental import pallas as pl
from jax.experimental.pallas import tpu as pltpu
from jax.experimental.pallas import tpu_sc as plsc

N = 10000
E = 320000
F = 128
FH = 32
FS = 4
H = 4
NCLASS = 64
NP = 10240
B = 128
NBLK = 157
EPT = NBLK * B
NTILE = 16
EPAD = NTILE * EPT
STRIPE = NP // NTILE
RB = 640
NEG = -1e30


def _proj_body(x_ref, w0_ref, w1_ref, al0_ref, ar0_ref, al1_ref, ar1_ref,
               ff_ref, el_ref, er_ref):
  xb = x_ref[...]
  for l in range(2):
    w_ref = (w0_ref, w1_ref)[l]
    al_ref = (al0_ref, al1_ref)[l]
    ar_ref = (ar0_ref, ar1_ref)[l]
    for h in range(H):
      f = jnp.dot(xb, w_ref[:, h * F:(h + 1) * F],
                  preferred_element_type=jnp.float32)
      for fh in range(FS):
        ff_ref[FS * (l * H + h) + fh] = f[:, fh * FH:(fh + 1) * FH]
      el_ref[l * H + h, :] = jnp.sum(f * al_ref[h, :][None, :], axis=-1)
      er_ref[l * H + h, :] = jnp.sum(f * ar_ref[h, :][None, :], axis=-1)


def _project(xp, W0, W1, al0, ar0, al1, ar1):
  full2 = lambda i: (0, 0)
  return pl.pallas_call(
      _proj_body,
      grid=(NP // RB,),
      in_specs=[
          pl.BlockSpec((RB, F), lambda i: (i, 0)),
          pl.BlockSpec((F, H * F), full2),
          pl.BlockSpec((F, H * F), full2),
          pl.BlockSpec((H, F), full2),
          pl.BlockSpec((H, F), full2),
          pl.BlockSpec((H, F), full2),
          pl.BlockSpec((H, F), full2),
      ],
      out_specs=[
          pl.BlockSpec((2 * H * FS, RB, FH), lambda i: (0, i, 0)),
          pl.BlockSpec((2 * H, RB), lambda i: (0, i)),
          pl.BlockSpec((2 * H, RB), lambda i: (0, i)),
      ],
      out_shape=[
          jax.ShapeDtypeStruct((2 * H * FS, NP, FH), jnp.float32),
          jax.ShapeDtypeStruct((2 * H, NP), jnp.float32),
          jax.ShapeDtypeStruct((2 * H, NP), jnp.float32),
      ],
  )(xp, W0, W1, al0, ar0, al1, ar1)


def _sc_body(srcs, dsts, ff, elcat, ercat,
             agg,
             src_all, dst_all, tab_a, tab_b, tab_c, abuf,
             in0, in1, out0, out1, bidx0, bidx1, sb0, sb1,
             den0, den1, acc,
             g0, g1, s0, s1, a0, a1):
  c = lax.axis_index("c")
  s = lax.axis_index("s")
  pltpu.sync_copy(srcs.at[s], src_all)
  pltpu.sync_copy(dsts.at[s], dst_all)

  zero16 = jnp.zeros((16,), jnp.float32)
  ins = (in0, in1)
  outs = (out0, out1)
  bidxs = (bidx0, bidx1)
  gsems = (g0, g1)
  ssems = (s0, s0)
  sbufs = (sb0, sb1)
  asems = (a0, a0)

  def _sval(j, k):
    sv = src_all[j, pl.ds(k * 16, 16)]
    dv = dst_all[j, pl.ds(k * 16, 16)]
    e = plsc.load_gather(tab_a, [sv]) + plsc.load_gather(tab_b, [dv])
    e = jnp.where(e > 0, e, 0.2 * e)
    return sv, dv, jnp.exp(e)

  def layer_body(li, carry):
    def _zt(i, carry1):
      tab_c[pl.ds(i * 16, 16)] = zero16
      return carry1
    lax.fori_loop(0, STRIPE // 16, _zt, 0)
    pltpu.sync_copy(tab_c.at[pl.ds(0, STRIPE)],
                    den0.at[pl.ds(s * STRIPE, STRIPE)])
    pltpu.sync_copy(tab_c.at[pl.ds(0, STRIPE)],
                    den1.at[pl.ds(s * STRIPE, STRIPE)])
    plsc.subcore_barrier()
    for hh in range(2):
      den = (den0, den1)[hh]
      lane = li * H + 2 * c + hh
      pltpu.sync_copy(elcat.at[pl.ds(lane * NP, NP)], tab_a)
      pltpu.sync_copy(ercat.at[pl.ds(lane * NP, NP)], tab_b)

      def stage1(j, b, do_wait):
        sb = sbufs[b]
        if do_wait:
          pltpu.make_async_copy(sb, den.at[dst_all.at[0]], asems[b]).wait()
        for k in range(8):
          _, _, sval = _sval(j, k)
          sb[pl.ds(k * 16, 16)] = sval
        pltpu.async_copy(sb, den.at[dst_all.at[j]], asems[b], add=True)

      stage1(0, 0, False)
      stage1(1, 1, False)

      def p1(g, carry1):
        stage1(2 * g, 0, True)
        stage1(2 * g + 1, 1, True)
        return carry1
      lax.fori_loop(1, NBLK // 2, p1, 0)
      stage1(NBLK - 1, 0, True)
      pltpu.make_async_copy(sb0, den.at[dst_all.at[0]], a0).wait()
      pltpu.make_async_copy(sb1, den.at[dst_all.at[0]], a0).wait()
    plsc.subcore_barrier()

    for hh in range(2):
      den = (den0, den1)[hh]
      lane = li * H + 2 * c + hh
      pltpu.sync_copy(elcat.at[pl.ds(lane * NP, NP)], tab_a)
      pltpu.sync_copy(ercat.at[pl.ds(lane * NP, NP)], tab_b)
      pltpu.sync_copy(den, tab_c)

      def ablk(j, carry1):
        for k in range(8):
          _, dv, sval = _sval(j, k)
          dval = plsc.load_gather(tab_c, [dv])
          abuf[pl.ds(j * B + k * 16, 16)] = sval / (dval + 1e-16)
        return carry1
      lax.fori_loop(0, NBLK, ablk, 0)

      for fh in range(FS):
        base = (FS * lane + fh) * NP
        bvec = jnp.full((16,), base, jnp.int32)

        def _zo(r, carry2):
          for k in range(FH // 16):
            out0[r, pl.ds(k * 16, 16)] = zero16
          return carry2
        lax.fori_loop(0, B, _zo, 0)
        for q in range(STRIPE // B):
          pltpu.sync_copy(out0, acc.at[pl.ds(s * STRIPE + q * B, B)])
        plsc.subcore_barrier()

        def _bidx(j, b):
          for k in range(8):
            bidxs[b][pl.ds(k * 16, 16)] = (
                src_all[j, pl.ds(k * 16, 16)] + bvec)

        def _scale(j, b):
          jb = j * B
          def rb(r4, carry2):
            r0 = r4 * 4
            for u in range(4):
              aspl = plsc.load_gather(
                  abuf, [jnp.full((16,), jb + r0 + u, jnp.int32)])
              for k in range(FH // 16):
                outs[b][r0 + u, pl.ds(k * 16, 16)] = (
                    ins[b][r0 + u, pl.ds(k * 16, 16)] * aspl)
            return carry2
          lax.fori_loop(0, B // 4, rb, 0)

        def stage2(j, b, do_swait, gnext):
          pltpu.make_async_copy(ff.at[bidxs[b]], ins[b], gsems[b]).wait()
          if do_swait:
            pltpu.make_async_copy(
                outs[b], acc.at[dst_all.at[0]], ssems[b]).wait()
          _scale(j, b)
          pltpu.async_copy(outs[b], acc.at[dst_all.at[j]], ssems[b],
                           add=True)
          if gnext:
            @pl.when(j + 2 < NBLK)
            def _():
              _bidx(j + 2, b)
              pltpu.async_copy(ff.at[bidxs[b]], ins[b], gsems[b])

        _bidx(0, 0)
        pltpu.async_copy(ff.at[bidx0], in0, g0)
        _bidx(1, 1)
        pltpu.async_copy(ff.at[bidx1], in1, g1)
        stage2(0, 0, False, True)
        stage2(1, 1, False, True)

        def p2(g, carry2):
          stage2(2 * g, 0, True, True)
          stage2(2 * g + 1, 1, True, True)
          return carry2
        lax.fori_loop(1, NBLK // 2, p2, 0)
        stage2(NBLK - 1, 0, True, False)
        pltpu.make_async_copy(out0, acc.at[dst_all.at[0]], s0).wait()
        pltpu.make_async_copy(out1, acc.at[dst_all.at[0]], s0).wait()

        plsc.subcore_barrier()
        row0 = base + s * STRIPE
        pltpu.sync_copy(acc.at[pl.ds(s * STRIPE, STRIPE)],
                        agg.at[pl.ds(row0, STRIPE)])
        plsc.subcore_barrier()
    return carry
  lax.fori_loop(0, 2, layer_body, 0)


def _sc_agg(srcs, dsts, ff, elcat, ercat):
  mesh = plsc.VectorSubcoreMesh(core_axis_name="c", subcore_axis_name="s")
  kfn = pl.kernel(
      _sc_body,
      out_type=jax.ShapeDtypeStruct((2 * H * FS * NP, FH), jnp.float32),
      mesh=mesh,
      compiler_params=pltpu.CompilerParams(needs_layout_passes=False,
                                           use_tc_tiling_on_sc=False),
      scratch_types=[
          pltpu.VMEM((NBLK, B), jnp.int32),
          pltpu.VMEM((NBLK, B), jnp.int32),
          pltpu.VMEM((NP,), jnp.float32),
          pltpu.VMEM((NP,), jnp.float32),
          pltpu.VMEM((NP,), jnp.float32),
          pltpu.VMEM((NBLK * B,), jnp.float32),
          pltpu.VMEM((B, FH), jnp.float32),
          pltpu.VMEM((B, FH), jnp.float32),
          pltpu.VMEM((B, FH), jnp.float32),
          pltpu.VMEM((B, FH), jnp.float32),
          pltpu.VMEM((B,), jnp.int32),
          pltpu.VMEM((B,), jnp.int32),
          pltpu.VMEM((B,), jnp.float32),
          pltpu.VMEM((B,), jnp.float32),
          pltpu.VMEM_SHARED((NP,), jnp.float32),
          pltpu.VMEM_SHARED((NP,), jnp.float32),
          pltpu.VMEM_SHARED((NP, FH), jnp.float32),
          pltpu.SemaphoreType.DMA,
          pltpu.SemaphoreType.DMA,
          pltpu.SemaphoreType.DMA,
          pltpu.SemaphoreType.DMA,
          pltpu.SemaphoreType.DMA,
          pltpu.SemaphoreType.DMA,
      ],
  )
  return kfn(srcs, dsts, ff, elcat, ercat)


def _out_body(agg_ref, wout_ref, bout_ref, o_ref):
  a = agg_ref[...]
  mls = []
  for l in range(2):
    parts = []
    for fh in range(FS):
      acc_fh = a[FS * (l * H + 0) + fh]
      for h in range(1, H):
        acc_fh = acc_fh + a[FS * (l * H + h) + fh]
      parts.append(acc_fh * 0.25)
    mls.append(jnp.concatenate(parts, axis=-1))
  h0 = jnp.maximum(mls[0], 0.0)
  h1 = jnp.maximum(h0 + mls[1], 0.0)
  lo = jnp.dot(h1, wout_ref[...], preferred_element_type=jnp.float32)
  lo = lo + bout_ref[...]
  z = lo - jnp.max(lo, axis=-1, keepdims=True)
  o_ref[...] = z - jnp.log(jnp.sum(jnp.exp(z), axis=-1, keepdims=True))


def _readout(agg, Wout, bout2):
  return pl.pallas_call(
      _out_body,
      grid=(NP // RB,),
      in_specs=[
          pl.BlockSpec((2 * H * FS, RB, FH), lambda i: (0, i, 0)),
          pl.BlockSpec((F, NCLASS), lambda i: (0, 0)),
          pl.BlockSpec((1, NCLASS), lambda i: (0, 0)),
      ],
      out_specs=pl.BlockSpec((RB, NCLASS), lambda i: (i, 0)),
      out_shape=jax.ShapeDtypeStruct((NP, NCLASS), jnp.float32),
  )(agg, Wout, bout2)


def kernel(x, edge_index, W0, al0, ar0, W1, al1, ar1, Wout, bout):
  src = edge_index[0].astype(jnp.int32)
  dst = edge_index[1].astype(jnp.int32)
  pad = EPAD - E
  srcs = jnp.concatenate([src, jnp.zeros((pad,), jnp.int32)])
  srcs = srcs.reshape(NTILE, NBLK, B)
  dsts = jnp.concatenate([dst, jnp.full((pad,), N, jnp.int32)])
  dsts = dsts.reshape(NTILE, NBLK, B)
  xp = jnp.pad(x, ((0, NP - N), (0, 0)))

  ff, el, er = _project(xp, W0, W1, al0, ar0, al1, ar1)
  er = er.at[:, N:].set(NEG)

  agg = _sc_agg(srcs, dsts, ff.reshape(2 * H * FS * NP, FH),
                el.reshape(2 * H * NP), er.reshape(2 * H * NP))

  out = _readout(agg.reshape(2 * H * FS, NP, FH), Wout,
                 bout.reshape(1, NCLASS).astype(jnp.float32))
  return out[:N]


# speedup vs baseline: 1.3469x; 1.2114x over previous
import jax
import jax.numpy as jnp
from jax import lax
from jax.experimental import pallas as pl
from jax.experimental.pallas import tpu as pltpu
from jax.experimental.pallas import tpu_sc as plsc

N = 10000
E = 320000
F = 128
FH = 32
FS = 4
H = 4
NCLASS = 64
NP = 10240
B = 128
NBLK = 157
EPT = NBLK * B
NTILE = 16
EPAD = NTILE * EPT
STRIPE = NP // NTILE
RB = 640
NEG = -1e30


def _proj_body(x_ref, w0_ref, w1_ref, al0_ref, ar0_ref, al1_ref, ar1_ref,
               ff_ref, el_ref, er_ref):
  xb = x_ref[...]
  for l in range(2):
    w_ref = (w0_ref, w1_ref)[l]
    al_ref = (al0_ref, al1_ref)[l]
    ar_ref = (ar0_ref, ar1_ref)[l]
    for h in range(H):
      f = jnp.dot(xb, w_ref[:, h * F:(h + 1) * F],
                  preferred_element_type=jnp.float32)
      for fh in range(FS):
        ff_ref[FS * (l * H + h) + fh] = f[:, fh * FH:(fh + 1) * FH]
      el_ref[l * H + h, :] = jnp.sum(f * al_ref[h, :][None, :], axis=-1)
      er_ref[l * H + h, :] = jnp.sum(f * ar_ref[h, :][None, :], axis=-1)


def _project(xp, W0, W1, al0, ar0, al1, ar1):
  full2 = lambda i: (0, 0)
  return pl.pallas_call(
      _proj_body,
      grid=(NP // RB,),
      in_specs=[
          pl.BlockSpec((RB, F), lambda i: (i, 0)),
          pl.BlockSpec((F, H * F), full2),
          pl.BlockSpec((F, H * F), full2),
          pl.BlockSpec((H, F), full2),
          pl.BlockSpec((H, F), full2),
          pl.BlockSpec((H, F), full2),
          pl.BlockSpec((H, F), full2),
      ],
      out_specs=[
          pl.BlockSpec((2 * H * FS, RB, FH), lambda i: (0, i, 0)),
          pl.BlockSpec((2 * H, RB), lambda i: (0, i)),
          pl.BlockSpec((2 * H, RB), lambda i: (0, i)),
      ],
      out_shape=[
          jax.ShapeDtypeStruct((2 * H * FS, NP, FH), jnp.float32),
          jax.ShapeDtypeStruct((2 * H, NP), jnp.float32),
          jax.ShapeDtypeStruct((2 * H, NP), jnp.float32),
      ],
  )(xp, W0, W1, al0, ar0, al1, ar1)


def _sc_body(srcs, dsts, ff, elcat, ercat,
             agg,
             src_all, dst_all, tab_a, tab_b, tab_c, abuf,
             in0, in1, in2, bidx0, bidx1, bidx2, sb0, sb1,
             den0, den1, acc,
             g0):
  c = lax.axis_index("c")
  s = lax.axis_index("s")
  pltpu.sync_copy(srcs.at[s], src_all)
  pltpu.sync_copy(dsts.at[s], dst_all)

  zero16 = jnp.zeros((16,), jnp.float32)
  ins = (in0, in1, in2)
  bidxs = (bidx0, bidx1, bidx2)
  gsems = (g0, g0, g0)
  sbufs = (sb0, sb1)

  def _sval(j, k):
    sv = src_all[j, pl.ds(k * 16, 16)]
    dv = dst_all[j, pl.ds(k * 16, 16)]
    e = plsc.load_gather(tab_a, [sv]) + plsc.load_gather(tab_b, [dv])
    e = jnp.where(e > 0, e, 0.2 * e)
    return sv, dv, jnp.exp(e)

  def layer_body(li, carry):
    def _zt(i, carry1):
      tab_c[pl.ds(i * 16, 16)] = zero16
      return carry1
    lax.fori_loop(0, STRIPE // 16, _zt, 0)
    pltpu.sync_copy(tab_c.at[pl.ds(0, STRIPE)],
                    den0.at[pl.ds(s * STRIPE, STRIPE)])
    pltpu.sync_copy(tab_c.at[pl.ds(0, STRIPE)],
                    den1.at[pl.ds(s * STRIPE, STRIPE)])
    plsc.subcore_barrier()
    for hh in range(2):
      den = (den0, den1)[hh]
      lane = li * H + 2 * c + hh
      pltpu.sync_copy(elcat.at[pl.ds(lane * NP, NP)], tab_a)
      pltpu.sync_copy(ercat.at[pl.ds(lane * NP, NP)], tab_b)

      def blk1(j, carry1):
        for k in range(8):
          _, _, sval = _sval(j, k)
          sb0[pl.ds(k * 16, 16)] = sval
        pltpu.sync_copy(sb0, den.at[dst_all.at[j]], add=True)
        return carry1
      lax.fori_loop(0, NBLK, blk1, 0)
    plsc.subcore_barrier()

    for hh in range(2):
      den = (den0, den1)[hh]
      lane = li * H + 2 * c + hh
      pltpu.sync_copy(elcat.at[pl.ds(lane * NP, NP)], tab_a)
      pltpu.sync_copy(ercat.at[pl.ds(lane * NP, NP)], tab_b)
      pltpu.sync_copy(den, tab_c)

      def ablk(j, carry1):
        for k in range(8):
          _, dv, sval = _sval(j, k)
          dval = plsc.load_gather(tab_c, [dv])
          abuf[pl.ds(j * B + k * 16, 16)] = sval / (dval + 1e-16)
        return carry1
      lax.fori_loop(0, NBLK, ablk, 0)

      def half_body(fh, carryh):
        base = (FS * lane + fh) * NP
        bvec = jnp.full((16,), base, jnp.int32)

        def _zo(r, carry2):
          for k in range(FH // 16):
            in0[r, pl.ds(k * 16, 16)] = zero16
          return carry2
        lax.fori_loop(0, B, _zo, 0)
        for q in range(STRIPE // B):
          pltpu.sync_copy(in0, acc.at[pl.ds(s * STRIPE + q * B, B)])
        plsc.subcore_barrier()

        def _bidx(j, b):
          for k in range(8):
            bidxs[b][pl.ds(k * 16, 16)] = (
                src_all[j, pl.ds(k * 16, 16)] + bvec)

        def _scale(j, b):
          jb = j * B
          def rb(r8, carry2):
            r0 = r8 * 8
            for u in range(8):
              aspl = plsc.load_gather(
                  abuf, [jnp.full((16,), jb + r0 + u, jnp.int32)])
              for k in range(FH // 16):
                ins[b][r0 + u, pl.ds(k * 16, 16)] = (
                    ins[b][r0 + u, pl.ds(k * 16, 16)] * aspl)
            return carry2
          lax.fori_loop(0, B // 8, rb, 0)

        def stage2(j, b, gnext):
          pltpu.make_async_copy(ff.at[bidxs[b]], ins[b], gsems[b]).wait()
          _scale(j, b)
          pltpu.sync_copy(ins[b], acc.at[dst_all.at[j]], add=True)
          if gnext:
            @pl.when(j + 3 < NBLK)
            def _():
              _bidx(j + 3, b)
              pltpu.async_copy(ff.at[bidxs[b]], ins[b], gsems[b])

        for b in range(3):
          _bidx(b, b)
          pltpu.async_copy(ff.at[bidxs[b]], ins[b], gsems[b])

        def p2(g, carry2):
          j = 3 * g
          stage2(j, 0, True)
          stage2(j + 1, 1, True)
          stage2(j + 2, 2, True)
          return carry2
        lax.fori_loop(0, NBLK // 3, p2, 0)
        stage2(NBLK - 1, 0, False)

        plsc.subcore_barrier()
        row0 = base + s * STRIPE
        pltpu.sync_copy(acc.at[pl.ds(s * STRIPE, STRIPE)],
                        agg.at[pl.ds(row0, STRIPE)])
        plsc.subcore_barrier()
        return carryh
      lax.fori_loop(0, FS, half_body, 0)
    return carry
  lax.fori_loop(0, 2, layer_body, 0)


def _sc_agg(srcs, dsts, ff, elcat, ercat):
  mesh = plsc.VectorSubcoreMesh(core_axis_name="c", subcore_axis_name="s")
  kfn = pl.kernel(
      _sc_body,
      out_type=jax.ShapeDtypeStruct((2 * H * FS * NP, FH), jnp.float32),
      mesh=mesh,
      compiler_params=pltpu.CompilerParams(needs_layout_passes=False,
                                           use_tc_tiling_on_sc=False),
      scratch_types=[
          pltpu.VMEM((NBLK, B), jnp.int32),
          pltpu.VMEM((NBLK, B), jnp.int32),
          pltpu.VMEM((NP,), jnp.float32),
          pltpu.VMEM((NP,), jnp.float32),
          pltpu.VMEM((NP,), jnp.float32),
          pltpu.VMEM((NBLK * B,), jnp.float32),
          pltpu.VMEM((B, FH), jnp.float32),
          pltpu.VMEM((B, FH), jnp.float32),
          pltpu.VMEM((B, FH), jnp.float32),
          pltpu.VMEM((B,), jnp.int32),
          pltpu.VMEM((B,), jnp.int32),
          pltpu.VMEM((B,), jnp.int32),
          pltpu.VMEM((B,), jnp.float32),
          pltpu.VMEM((B,), jnp.float32),
          pltpu.VMEM_SHARED((NP,), jnp.float32),
          pltpu.VMEM_SHARED((NP,), jnp.float32),
          pltpu.VMEM_SHARED((NP, FH), jnp.float32),
          pltpu.SemaphoreType.DMA,
      ],
  )
  return kfn(srcs, dsts, ff, elcat, ercat)


def _out_body(agg_ref, wout_ref, bout_ref, o_ref):
  a = agg_ref[...]
  mls = []
  for l in range(2):
    parts = []
    for fh in range(FS):
      acc_fh = a[FS * (l * H + 0) + fh]
      for h in range(1, H):
        acc_fh = acc_fh + a[FS * (l * H + h) + fh]
      parts.append(acc_fh * 0.25)
    mls.append(jnp.concatenate(parts, axis=-1))
  h0 = jnp.maximum(mls[0], 0.0)
  h1 = jnp.maximum(h0 + mls[1], 0.0)
  lo = jnp.dot(h1, wout_ref[...], preferred_element_type=jnp.float32)
  lo = lo + bout_ref[...]
  z = lo - jnp.max(lo, axis=-1, keepdims=True)
  o_ref[...] = z - jnp.log(jnp.sum(jnp.exp(z), axis=-1, keepdims=True))


def _readout(agg, Wout, bout2):
  return pl.pallas_call(
      _out_body,
      grid=(NP // RB,),
      in_specs=[
          pl.BlockSpec((2 * H * FS, RB, FH), lambda i: (0, i, 0)),
          pl.BlockSpec((F, NCLASS), lambda i: (0, 0)),
          pl.BlockSpec((1, NCLASS), lambda i: (0, 0)),
      ],
      out_specs=pl.BlockSpec((RB, NCLASS), lambda i: (i, 0)),
      out_shape=jax.ShapeDtypeStruct((NP, NCLASS), jnp.float32),
  )(agg, Wout, bout2)


def kernel(x, edge_index, W0, al0, ar0, W1, al1, ar1, Wout, bout):
  src = edge_index[0].astype(jnp.int32)
  dst = edge_index[1].astype(jnp.int32)
  pad = EPAD - E
  srcs = jnp.concatenate([src, jnp.zeros((pad,), jnp.int32)])
  srcs = srcs.reshape(NTILE, NBLK, B)
  dsts = jnp.concatenate([dst, jnp.full((pad,), N, jnp.int32)])
  dsts = dsts.reshape(NTILE, NBLK, B)
  xp = jnp.pad(x, ((0, NP - N), (0, 0)))

  ff, el, er = _project(xp, W0, W1, al0, ar0, al1, ar1)
  er = er.at[:, N:].set(NEG)

  agg = _sc_agg(srcs, dsts, ff.reshape(2 * H * FS * NP, FH),
                el.reshape(2 * H * NP), er.reshape(2 * H * NP))

  out = _readout(agg.reshape(2 * H * FS, NP, FH), Wout,
                 bout.reshape(1, NCLASS).astype(jnp.float32))
  return out[:N]


# speedup vs baseline: 1.3682x; 1.0158x over previous
import jax
import jax.numpy as jnp
from jax import lax
from jax.experimental import pallas as pl
from jax.experimental.pallas import tpu as pltpu
from jax.experimental.pallas import tpu_sc as plsc

N = 10000
E = 320000
F = 128
FH = 32
FS = 4
H = 4
NCLASS = 64
NP = 10240
B = 128
NBLK = 157
EPT = NBLK * B
NTILE = 16
EPAD = NTILE * EPT
STRIPE = NP // NTILE
RB = 640
NEG = -1e30


def _proj_body(x_ref, w0_ref, w1_ref, al0_ref, ar0_ref, al1_ref, ar1_ref,
               ff_ref, el_ref, er_ref):
  xb = x_ref[...]
  for l in range(2):
    w_ref = (w0_ref, w1_ref)[l]
    al_ref = (al0_ref, al1_ref)[l]
    ar_ref = (ar0_ref, ar1_ref)[l]
    for h in range(H):
      f = jnp.dot(xb, w_ref[:, h * F:(h + 1) * F],
                  preferred_element_type=jnp.float32)
      for fh in range(FS):
        ff_ref[FS * (l * H + h) + fh] = f[:, fh * FH:(fh + 1) * FH]
      el_ref[l * H + h, :] = jnp.sum(f * al_ref[h, :][None, :], axis=-1)
      er_ref[l * H + h, :] = jnp.sum(f * ar_ref[h, :][None, :], axis=-1)


def _project(xp, W0, W1, al0, ar0, al1, ar1):
  full2 = lambda i: (0, 0)
  return pl.pallas_call(
      _proj_body,
      grid=(NP // RB,),
      in_specs=[
          pl.BlockSpec((RB, F), lambda i: (i, 0)),
          pl.BlockSpec((F, H * F), full2),
          pl.BlockSpec((F, H * F), full2),
          pl.BlockSpec((H, F), full2),
          pl.BlockSpec((H, F), full2),
          pl.BlockSpec((H, F), full2),
          pl.BlockSpec((H, F), full2),
      ],
      out_specs=[
          pl.BlockSpec((2 * H * FS, RB, FH), lambda i: (0, i, 0)),
          pl.BlockSpec((2 * H, RB), lambda i: (0, i)),
          pl.BlockSpec((2 * H, RB), lambda i: (0, i)),
      ],
      out_shape=[
          jax.ShapeDtypeStruct((2 * H * FS, NP, FH), jnp.float32),
          jax.ShapeDtypeStruct((2 * H, NP), jnp.float32),
          jax.ShapeDtypeStruct((2 * H, NP), jnp.float32),
      ],
  )(xp, W0, W1, al0, ar0, al1, ar1)


def _sc_body(srcs, dsts, ff, elcat, ercat,
             agg,
             src_all, dst_all, tab_a, tab_b, tab_c, abuf,
             in0, in1, in2, bidx0, bidx1, bidx2, sb0, sb1,
             den0, den1, acc,
             g0):
  c = lax.axis_index("c")
  s = lax.axis_index("s")
  pltpu.sync_copy(srcs.at[s], src_all)
  pltpu.sync_copy(dsts.at[s], dst_all)

  zero16 = jnp.zeros((16,), jnp.float32)
  ins = (in0, in1, in2)
  bidxs = (bidx0, bidx1, bidx2)
  gsems = (g0, g0, g0)
  sbufs = (sb0, sb1)

  def _sval(j, k):
    sv = src_all[j, pl.ds(k * 16, 16)]
    dv = dst_all[j, pl.ds(k * 16, 16)]
    e = plsc.load_gather(tab_a, [sv]) + plsc.load_gather(tab_b, [dv])
    e = jnp.where(e > 0, e, 0.2 * e)
    return sv, dv, jnp.exp(e)

  def layer_body(li, carry):
    def _zt(i, carry1):
      tab_c[pl.ds(i * 16, 16)] = zero16
      return carry1
    lax.fori_loop(0, STRIPE // 16, _zt, 0)
    pltpu.sync_copy(tab_c.at[pl.ds(0, STRIPE)],
                    den0.at[pl.ds(s * STRIPE, STRIPE)])
    pltpu.sync_copy(tab_c.at[pl.ds(0, STRIPE)],
                    den1.at[pl.ds(s * STRIPE, STRIPE)])
    plsc.subcore_barrier()
    for hh in range(2):
      den = (den0, den1)[hh]
      lane = li * H + 2 * c + hh
      pltpu.sync_copy(elcat.at[pl.ds(lane * NP, NP)], tab_a)
      pltpu.sync_copy(ercat.at[pl.ds(lane * NP, NP)], tab_b)

      def blk1(j, carry1):
        for k in range(8):
          _, _, sval = _sval(j, k)
          abuf[pl.ds(j * B + k * 16, 16)] = sval
        pltpu.sync_copy(abuf.at[pl.ds(j * B, B)], den.at[dst_all.at[j]],
                        add=True)
        return carry1
      lax.fori_loop(0, NBLK, blk1, 0)
      plsc.subcore_barrier()

      pltpu.sync_copy(den, tab_c)

      def ablk(j, carry1):
        for k in range(8):
          dv = dst_all[j, pl.ds(k * 16, 16)]
          dval = plsc.load_gather(tab_c, [dv])
          sl = pl.ds(j * B + k * 16, 16)
          abuf[sl] = abuf[sl] / (dval + 1e-16)
        return carry1
      lax.fori_loop(0, NBLK, ablk, 0)

      def half_body(fh, carryh):
        base = (FS * lane + fh) * NP
        bvec = jnp.full((16,), base, jnp.int32)

        def _zo(r, carry2):
          for k in range(FH // 16):
            in0[r, pl.ds(k * 16, 16)] = zero16
          return carry2
        lax.fori_loop(0, B, _zo, 0)
        for q in range(STRIPE // B):
          pltpu.sync_copy(in0, acc.at[pl.ds(s * STRIPE + q * B, B)])
        plsc.subcore_barrier()

        def _bidx(j, b):
          for k in range(8):
            bidxs[b][pl.ds(k * 16, 16)] = (
                src_all[j, pl.ds(k * 16, 16)] + bvec)

        def _scale(j, b):
          jb = j * B
          def rb(r8, carry2):
            r0 = r8 * 8
            for u in range(8):
              aspl = plsc.load_gather(
                  abuf, [jnp.full((16,), jb + r0 + u, jnp.int32)])
              for k in range(FH // 16):
                ins[b][r0 + u, pl.ds(k * 16, 16)] = (
                    ins[b][r0 + u, pl.ds(k * 16, 16)] * aspl)
            return carry2
          lax.fori_loop(0, B // 8, rb, 0)

        def stage2(j, b, gnext):
          pltpu.make_async_copy(ff.at[bidxs[b]], ins[b], gsems[b]).wait()
          _scale(j, b)
          pltpu.sync_copy(ins[b], acc.at[dst_all.at[j]], add=True)
          if gnext:
            @pl.when(j + 3 < NBLK)
            def _():
              _bidx(j + 3, b)
              pltpu.async_copy(ff.at[bidxs[b]], ins[b], gsems[b])

        for b in range(3):
          _bidx(b, b)
          pltpu.async_copy(ff.at[bidxs[b]], ins[b], gsems[b])

        def p2(g, carry2):
          j = 3 * g
          stage2(j, 0, True)
          stage2(j + 1, 1, True)
          stage2(j + 2, 2, True)
          return carry2
        lax.fori_loop(0, NBLK // 3, p2, 0)
        stage2(NBLK - 1, 0, False)

        plsc.subcore_barrier()
        row0 = base + s * STRIPE
        pltpu.sync_copy(acc.at[pl.ds(s * STRIPE, STRIPE)],
                        agg.at[pl.ds(row0, STRIPE)])
        plsc.subcore_barrier()
        return carryh
      lax.fori_loop(0, FS, half_body, 0)
    return carry
  lax.fori_loop(0, 2, layer_body, 0)


def _sc_agg(srcs, dsts, ff, elcat, ercat):
  mesh = plsc.VectorSubcoreMesh(core_axis_name="c", subcore_axis_name="s")
  kfn = pl.kernel(
      _sc_body,
      out_type=jax.ShapeDtypeStruct((2 * H * FS * NP, FH), jnp.float32),
      mesh=mesh,
      compiler_params=pltpu.CompilerParams(needs_layout_passes=False,
                                           use_tc_tiling_on_sc=False),
      scratch_types=[
          pltpu.VMEM((NBLK, B), jnp.int32),
          pltpu.VMEM((NBLK, B), jnp.int32),
          pltpu.VMEM((NP,), jnp.float32),
          pltpu.VMEM((NP,), jnp.float32),
          pltpu.VMEM((NP,), jnp.float32),
          pltpu.VMEM((NBLK * B,), jnp.float32),
          pltpu.VMEM((B, FH), jnp.float32),
          pltpu.VMEM((B, FH), jnp.float32),
          pltpu.VMEM((B, FH), jnp.float32),
          pltpu.VMEM((B,), jnp.int32),
          pltpu.VMEM((B,), jnp.int32),
          pltpu.VMEM((B,), jnp.int32),
          pltpu.VMEM((B,), jnp.float32),
          pltpu.VMEM((B,), jnp.float32),
          pltpu.VMEM_SHARED((NP,), jnp.float32),
          pltpu.VMEM_SHARED((NP,), jnp.float32),
          pltpu.VMEM_SHARED((NP, FH), jnp.float32),
          pltpu.SemaphoreType.DMA,
      ],
  )
  return kfn(srcs, dsts, ff, elcat, ercat)


def _out_body(agg_ref, wout_ref, bout_ref, o_ref):
  a = agg_ref[...]
  mls = []
  for l in range(2):
    parts = []
    for fh in range(FS):
      acc_fh = a[FS * (l * H + 0) + fh]
      for h in range(1, H):
        acc_fh = acc_fh + a[FS * (l * H + h) + fh]
      parts.append(acc_fh * 0.25)
    mls.append(jnp.concatenate(parts, axis=-1))
  h0 = jnp.maximum(mls[0], 0.0)
  h1 = jnp.maximum(h0 + mls[1], 0.0)
  lo = jnp.dot(h1, wout_ref[...], preferred_element_type=jnp.float32)
  lo = lo + bout_ref[...]
  z = lo - jnp.max(lo, axis=-1, keepdims=True)
  o_ref[...] = z - jnp.log(jnp.sum(jnp.exp(z), axis=-1, keepdims=True))


def _readout(agg, Wout, bout2):
  return pl.pallas_call(
      _out_body,
      grid=(NP // RB,),
      in_specs=[
          pl.BlockSpec((2 * H * FS, RB, FH), lambda i: (0, i, 0)),
          pl.BlockSpec((F, NCLASS), lambda i: (0, 0)),
          pl.BlockSpec((1, NCLASS), lambda i: (0, 0)),
      ],
      out_specs=pl.BlockSpec((RB, NCLASS), lambda i: (i, 0)),
      out_shape=jax.ShapeDtypeStruct((NP, NCLASS), jnp.float32),
  )(agg, Wout, bout2)


def kernel(x, edge_index, W0, al0, ar0, W1, al1, ar1, Wout, bout):
  src = edge_index[0].astype(jnp.int32)
  dst = edge_index[1].astype(jnp.int32)
  pad = EPAD - E
  srcs = jnp.concatenate([src, jnp.zeros((pad,), jnp.int32)])
  srcs = srcs.reshape(NTILE, NBLK, B)
  dsts = jnp.concatenate([dst, jnp.full((pad,), N, jnp.int32)])
  dsts = dsts.reshape(NTILE, NBLK, B)
  xp = jnp.pad(x, ((0, NP - N), (0, 0)))

  ff, el, er = _project(xp, W0, W1, al0, ar0, al1, ar1)
  er = er.at[:, N:].set(NEG)

  agg = _sc_agg(srcs, dsts, ff.reshape(2 * H * FS * NP, FH),
                el.reshape(2 * H * NP), er.reshape(2 * H * NP))

  out = _readout(agg.reshape(2 * H * FS, NP, FH), Wout,
                 bout.reshape(1, NCLASS).astype(jnp.float32))
  return out[:N]


# speedup vs baseline: 1.3930x; 1.0181x over previous
import jax
import jax.numpy as jnp
from jax import lax
from jax.experimental import pallas as pl
from jax.experimental.pallas import tpu as pltpu
from jax.experimental.pallas import tpu_sc as plsc

N = 10000
E = 320000
F = 128
FH = 32
FS = 4
H = 4
NCLASS = 64
NP = 10240
B = 128
NBLK = 157
EPT = NBLK * B
NTILE = 16
EPAD = NTILE * EPT
STRIPE = NP // NTILE
RB = 640
NEG = -1e30


def _proj_body(x_ref, w0_ref, w1_ref, al0_ref, ar0_ref, al1_ref, ar1_ref,
               ff_ref, el_ref, er_ref):
  xb = x_ref[...]
  for l in range(2):
    w_ref = (w0_ref, w1_ref)[l]
    al_ref = (al0_ref, al1_ref)[l]
    ar_ref = (ar0_ref, ar1_ref)[l]
    for h in range(H):
      f = jnp.dot(xb, w_ref[:, h * F:(h + 1) * F],
                  preferred_element_type=jnp.float32)
      for fh in range(FS):
        ff_ref[FS * (l * H + h) + fh] = f[:, fh * FH:(fh + 1) * FH]
      el_ref[l * H + h, :] = jnp.sum(f * al_ref[h, :][None, :], axis=-1)
      er_ref[l * H + h, :] = jnp.sum(f * ar_ref[h, :][None, :], axis=-1)


def _project(xp, W0, W1, al0, ar0, al1, ar1):
  full2 = lambda i: (0, 0)
  return pl.pallas_call(
      _proj_body,
      grid=(NP // RB,),
      in_specs=[
          pl.BlockSpec((RB, F), lambda i: (i, 0)),
          pl.BlockSpec((F, H * F), full2),
          pl.BlockSpec((F, H * F), full2),
          pl.BlockSpec((H, F), full2),
          pl.BlockSpec((H, F), full2),
          pl.BlockSpec((H, F), full2),
          pl.BlockSpec((H, F), full2),
      ],
      out_specs=[
          pl.BlockSpec((2 * H * FS, RB, FH), lambda i: (0, i, 0)),
          pl.BlockSpec((2 * H, RB), lambda i: (0, i)),
          pl.BlockSpec((2 * H, RB), lambda i: (0, i)),
      ],
      out_shape=[
          jax.ShapeDtypeStruct((2 * H * FS, NP, FH), jnp.float32),
          jax.ShapeDtypeStruct((2 * H, NP), jnp.float32),
          jax.ShapeDtypeStruct((2 * H, NP), jnp.float32),
      ],
  )(xp, W0, W1, al0, ar0, al1, ar1)


def _sc_body(srcs, dsts, ff, elcat, ercat,
             agg,
             src_all, dst_all, tab_a, tab_b, tab_c, abuf,
             in0, in1, in2, bidx0, bidx1, bidx2, sb0, sb1,
             den0, den1, acc,
             g0, a0):
  c = lax.axis_index("c")
  s = lax.axis_index("s")
  pltpu.sync_copy(srcs.at[s], src_all)
  pltpu.sync_copy(dsts.at[s], dst_all)

  zero16 = jnp.zeros((16,), jnp.float32)
  ins = (in0, in1, in2)
  bidxs = (bidx0, bidx1, bidx2)
  gsems = (g0, g0, g0)
  sbufs = (sb0, sb1)

  def _sval(j, k):
    sv = src_all[j, pl.ds(k * 16, 16)]
    dv = dst_all[j, pl.ds(k * 16, 16)]
    e = plsc.load_gather(tab_a, [sv]) + plsc.load_gather(tab_b, [dv])
    e = jnp.where(e > 0, e, 0.2 * e)
    return sv, dv, jnp.exp(e)

  def layer_body(li, carry):
    def _zt(i, carry1):
      tab_c[pl.ds(i * 16, 16)] = zero16
      return carry1
    lax.fori_loop(0, STRIPE // 16, _zt, 0)
    pltpu.sync_copy(tab_c.at[pl.ds(0, STRIPE)],
                    den0.at[pl.ds(s * STRIPE, STRIPE)])
    pltpu.sync_copy(tab_c.at[pl.ds(0, STRIPE)],
                    den1.at[pl.ds(s * STRIPE, STRIPE)])
    plsc.subcore_barrier()
    for hh in range(2):
      den = (den0, den1)[hh]
      lane = li * H + 2 * c + hh
      pltpu.sync_copy(elcat.at[pl.ds(lane * NP, NP)], tab_a)
      pltpu.sync_copy(ercat.at[pl.ds(lane * NP, NP)], tab_b)

      def blk1(j, carry1):
        for k in range(8):
          _, _, sval = _sval(j, k)
          abuf[pl.ds(j * B + k * 16, 16)] = sval
        pltpu.async_copy(abuf.at[pl.ds(j * B, B)], den.at[dst_all.at[j]],
                         a0, add=True)
        return carry1
      lax.fori_loop(0, NBLK, blk1, 0)

      def drain1(j, carry1):
        pltpu.make_async_copy(abuf.at[pl.ds(0, B)], den.at[dst_all.at[0]],
                              a0).wait()
        return carry1
      lax.fori_loop(0, NBLK, drain1, 0)
      plsc.subcore_barrier()

      pltpu.sync_copy(den, tab_c)

      def ablk(j, carry1):
        for k in range(8):
          dv = dst_all[j, pl.ds(k * 16, 16)]
          dval = plsc.load_gather(tab_c, [dv])
          sl = pl.ds(j * B + k * 16, 16)
          abuf[sl] = abuf[sl] / (dval + 1e-16)
        return carry1
      lax.fori_loop(0, NBLK, ablk, 0)

      def half_body(fh, carryh):
        base = (FS * lane + fh) * NP
        bvec = jnp.full((16,), base, jnp.int32)

        def _zo(r, carry2):
          for k in range(FH // 16):
            in0[r, pl.ds(k * 16, 16)] = zero16
          return carry2
        lax.fori_loop(0, B, _zo, 0)
        for q in range(STRIPE // B):
          pltpu.sync_copy(in0, acc.at[pl.ds(s * STRIPE + q * B, B)])
        plsc.subcore_barrier()

        def _bidx(j, b):
          for k in range(8):
            bidxs[b][pl.ds(k * 16, 16)] = (
                src_all[j, pl.ds(k * 16, 16)] + bvec)

        def _scale(j, b):
          jb = j * B
          def rb(r8, carry2):
            r0 = r8 * 8
            for u in range(8):
              aspl = plsc.load_gather(
                  abuf, [jnp.full((16,), jb + r0 + u, jnp.int32)])
              for k in range(FH // 16):
                ins[b][r0 + u, pl.ds(k * 16, 16)] = (
                    ins[b][r0 + u, pl.ds(k * 16, 16)] * aspl)
            return carry2
          lax.fori_loop(0, B // 8, rb, 0)

        def stage2(j, b, gnext):
          pltpu.make_async_copy(ff.at[bidxs[b]], ins[b], gsems[b]).wait()
          _scale(j, b)
          pltpu.sync_copy(ins[b], acc.at[dst_all.at[j]], add=True)
          if gnext:
            @pl.when(j + 3 < NBLK)
            def _():
              _bidx(j + 3, b)
              pltpu.async_copy(ff.at[bidxs[b]], ins[b], gsems[b])

        for b in range(3):
          _bidx(b, b)
          pltpu.async_copy(ff.at[bidxs[b]], ins[b], gsems[b])

        def p2(g, carry2):
          j = 3 * g
          stage2(j, 0, True)
          stage2(j + 1, 1, True)
          stage2(j + 2, 2, True)
          return carry2
        lax.fori_loop(0, NBLK // 3, p2, 0)
        stage2(NBLK - 1, 0, False)

        plsc.subcore_barrier()
        row0 = base + s * STRIPE
        pltpu.sync_copy(acc.at[pl.ds(s * STRIPE, STRIPE)],
                        agg.at[pl.ds(row0, STRIPE)])
        plsc.subcore_barrier()
        return carryh
      lax.fori_loop(0, FS, half_body, 0)
    return carry
  lax.fori_loop(0, 2, layer_body, 0)


def _sc_agg(srcs, dsts, ff, elcat, ercat):
  mesh = plsc.VectorSubcoreMesh(core_axis_name="c", subcore_axis_name="s")
  kfn = pl.kernel(
      _sc_body,
      out_type=jax.ShapeDtypeStruct((2 * H * FS * NP, FH), jnp.float32),
      mesh=mesh,
      compiler_params=pltpu.CompilerParams(needs_layout_passes=False,
                                           use_tc_tiling_on_sc=False),
      scratch_types=[
          pltpu.VMEM((NBLK, B), jnp.int32),
          pltpu.VMEM((NBLK, B), jnp.int32),
          pltpu.VMEM((NP,), jnp.float32),
          pltpu.VMEM((NP,), jnp.float32),
          pltpu.VMEM((NP,), jnp.float32),
          pltpu.VMEM((NBLK * B,), jnp.float32),
          pltpu.VMEM((B, FH), jnp.float32),
          pltpu.VMEM((B, FH), jnp.float32),
          pltpu.VMEM((B, FH), jnp.float32),
          pltpu.VMEM((B,), jnp.int32),
          pltpu.VMEM((B,), jnp.int32),
          pltpu.VMEM((B,), jnp.int32),
          pltpu.VMEM((B,), jnp.float32),
          pltpu.VMEM((B,), jnp.float32),
          pltpu.VMEM_SHARED((NP,), jnp.float32),
          pltpu.VMEM_SHARED((NP,), jnp.float32),
          pltpu.VMEM_SHARED((NP, FH), jnp.float32),
          pltpu.SemaphoreType.DMA,
          pltpu.SemaphoreType.DMA,
      ],
  )
  return kfn(srcs, dsts, ff, elcat, ercat)


def _out_body(agg_ref, wout_ref, bout_ref, o_ref):
  a = agg_ref[...]
  mls = []
  for l in range(2):
    parts = []
    for fh in range(FS):
      acc_fh = a[FS * (l * H + 0) + fh]
      for h in range(1, H):
        acc_fh = acc_fh + a[FS * (l * H + h) + fh]
      parts.append(acc_fh * 0.25)
    mls.append(jnp.concatenate(parts, axis=-1))
  h0 = jnp.maximum(mls[0], 0.0)
  h1 = jnp.maximum(h0 + mls[1], 0.0)
  lo = jnp.dot(h1, wout_ref[...], preferred_element_type=jnp.float32)
  lo = lo + bout_ref[...]
  z = lo - jnp.max(lo, axis=-1, keepdims=True)
  o_ref[...] = z - jnp.log(jnp.sum(jnp.exp(z), axis=-1, keepdims=True))


def _readout(agg, Wout, bout2):
  return pl.pallas_call(
      _out_body,
      grid=(NP // RB,),
      in_specs=[
          pl.BlockSpec((2 * H * FS, RB, FH), lambda i: (0, i, 0)),
          pl.BlockSpec((F, NCLASS), lambda i: (0, 0)),
          pl.BlockSpec((1, NCLASS), lambda i: (0, 0)),
      ],
      out_specs=pl.BlockSpec((RB, NCLASS), lambda i: (i, 0)),
      out_shape=jax.ShapeDtypeStruct((NP, NCLASS), jnp.float32),
  )(agg, Wout, bout2)


def kernel(x, edge_index, W0, al0, ar0, W1, al1, ar1, Wout, bout):
  src = edge_index[0].astype(jnp.int32)
  dst = edge_index[1].astype(jnp.int32)
  pad = EPAD - E
  srcs = jnp.concatenate([src, jnp.zeros((pad,), jnp.int32)])
  srcs = srcs.reshape(NTILE, NBLK, B)
  dsts = jnp.concatenate([dst, jnp.full((pad,), N, jnp.int32)])
  dsts = dsts.reshape(NTILE, NBLK, B)
  xp = jnp.pad(x, ((0, NP - N), (0, 0)))

  ff, el, er = _project(xp, W0, W1, al0, ar0, al1, ar1)
  er = er.at[:, N:].set(NEG)

  agg = _sc_agg(srcs, dsts, ff.reshape(2 * H * FS * NP, FH),
                el.reshape(2 * H * NP), er.reshape(2 * H * NP))

  out = _readout(agg.reshape(2 * H * FS, NP, FH), Wout,
                 bout.reshape(1, NCLASS).astype(jnp.float32))
  return out[:N]


# speedup vs baseline: 2.1628x; 1.5526x over previous
import jax
import jax.numpy as jnp
from jax import lax
from jax.experimental import pallas as pl
from jax.experimental.pallas import tpu as pltpu
from jax.experimental.pallas import tpu_sc as plsc

N = 10000
E = 320000
F = 128
FH = 32
FS = 4
H = 4
NCLASS = 64
NP = 10240
B = 128
NBLK = 157
EPT = NBLK * B
NTILE = 16
EPAD = NTILE * EPT
STRIPE = NP // NTILE
RB = 640
NEG = -1e30


def _proj_body(x_ref, w0_ref, w1_ref, al0_ref, ar0_ref, al1_ref, ar1_ref,
               ff_ref, el_ref, er_ref):
  xb = x_ref[...]
  for l in range(2):
    w_ref = (w0_ref, w1_ref)[l]
    al_ref = (al0_ref, al1_ref)[l]
    ar_ref = (ar0_ref, ar1_ref)[l]
    for h in range(H):
      f = jnp.dot(xb, w_ref[:, h * F:(h + 1) * F],
                  preferred_element_type=jnp.float32)
      for fh in range(FS):
        ff_ref[FS * (l * H + h) + fh] = f[:, fh * FH:(fh + 1) * FH]
      el_ref[l * H + h, :] = jnp.sum(f * al_ref[h, :][None, :], axis=-1)
      er_ref[l * H + h, :] = jnp.sum(f * ar_ref[h, :][None, :], axis=-1)


def _project(xp, W0, W1, al0, ar0, al1, ar1):
  full2 = lambda i: (0, 0)
  return pl.pallas_call(
      _proj_body,
      grid=(NP // RB,),
      in_specs=[
          pl.BlockSpec((RB, F), lambda i: (i, 0)),
          pl.BlockSpec((F, H * F), full2),
          pl.BlockSpec((F, H * F), full2),
          pl.BlockSpec((H, F), full2),
          pl.BlockSpec((H, F), full2),
          pl.BlockSpec((H, F), full2),
          pl.BlockSpec((H, F), full2),
      ],
      out_specs=[
          pl.BlockSpec((2 * H * FS, RB, FH), lambda i: (0, i, 0)),
          pl.BlockSpec((2 * H, RB), lambda i: (0, i)),
          pl.BlockSpec((2 * H, RB), lambda i: (0, i)),
      ],
      out_shape=[
          jax.ShapeDtypeStruct((2 * H * FS, NP, FH), jnp.float32),
          jax.ShapeDtypeStruct((2 * H, NP), jnp.float32),
          jax.ShapeDtypeStruct((2 * H, NP), jnp.float32),
      ],
  )(xp, W0, W1, al0, ar0, al1, ar1)


def _sc_body(srcs, dsts, ff, elcat, ercat,
             agg,
             src_all, dst_all, tab_a, tab_b, tab_c, abuf,
             in0, in1, in2, bidx0, bidx1, bidx2, sb0, sb1,
             den0, den1, acc,
             g0, a0):
  c = lax.axis_index("c")
  s = lax.axis_index("s")
  pltpu.sync_copy(srcs.at[s], src_all)
  pltpu.sync_copy(dsts.at[s], dst_all)

  zero16 = jnp.zeros((16,), jnp.float32)
  ins = (in0, in1, in2)
  bidxs = (bidx0, bidx1, bidx2)
  gsems = (g0, g0, g0)
  sbufs = (sb0, sb1)

  def _sval(j, k):
    sv = src_all[j, pl.ds(k * 16, 16)]
    dv = dst_all[j, pl.ds(k * 16, 16)]
    e = plsc.load_gather(tab_a, [sv]) + plsc.load_gather(tab_b, [dv])
    e = jnp.where(e > 0, e, 0.2 * e)
    return sv, dv, jnp.exp(e)

  def layer_body(li, carry):
    def _zt(i, carry1):
      tab_c[pl.ds(i * 16, 16)] = zero16
      return carry1
    lax.fori_loop(0, STRIPE // 16, _zt, 0)
    pltpu.sync_copy(tab_c.at[pl.ds(0, STRIPE)],
                    den0.at[pl.ds(s * STRIPE, STRIPE)])
    pltpu.sync_copy(tab_c.at[pl.ds(0, STRIPE)],
                    den1.at[pl.ds(s * STRIPE, STRIPE)])
    plsc.subcore_barrier()
    for hh in range(2):
      den = (den0, den1)[hh]
      lane = li * H + 2 * c + hh
      pltpu.sync_copy(elcat.at[pl.ds(lane * NP, NP)], tab_a)
      pltpu.sync_copy(ercat.at[pl.ds(lane * NP, NP)], tab_b)

      def blk1(j, carry1):
        for k in range(8):
          _, _, sval = _sval(j, k)
          abuf[pl.ds(j * B + k * 16, 16)] = sval
        pltpu.async_copy(abuf.at[pl.ds(j * B, B)], den.at[dst_all.at[j]],
                         a0, add=True)
        return carry1
      lax.fori_loop(0, NBLK, blk1, 0)

      def drain1(j, carry1):
        pltpu.make_async_copy(abuf.at[pl.ds(0, B)], den.at[dst_all.at[0]],
                              a0).wait()
        return carry1
      lax.fori_loop(0, NBLK, drain1, 0)
      plsc.subcore_barrier()

      pltpu.sync_copy(den, tab_c)

      def ablk(j, carry1):
        for k in range(8):
          dv = dst_all[j, pl.ds(k * 16, 16)]
          dval = plsc.load_gather(tab_c, [dv])
          sl = pl.ds(j * B + k * 16, 16)
          abuf[sl] = abuf[sl] / (dval + 1e-16)
        return carry1
      lax.fori_loop(0, NBLK, ablk, 0)

      def half_body(fh, carryh):
        base = (FS * lane + fh) * NP
        bvec = jnp.full((16,), base, jnp.int32)

        def _zo(r, carry2):
          for k in range(FH // 16):
            in0[r, pl.ds(k * 16, 16)] = zero16
          return carry2
        lax.fori_loop(0, B, _zo, 0)
        for q in range(STRIPE // B):
          pltpu.sync_copy(in0, acc.at[pl.ds(s * STRIPE + q * B, B)])
        plsc.subcore_barrier()

        def _bidx(j, b):
          for k in range(8):
            bidxs[b][pl.ds(k * 16, 16)] = (
                src_all[j, pl.ds(k * 16, 16)] + bvec)

        def _scale(j, b):
          jb = j * B
          def rb(r16, carry2):
            r0 = r16 * 16
            av = abuf[pl.ds(jb + r0, 16)]
            for u in range(16):
              aspl = lax.gather(
                  av, jnp.full((16, 1), u, jnp.int32),
                  lax.GatherDimensionNumbers(
                      offset_dims=(), collapsed_slice_dims=(0,),
                      start_index_map=(0,)),
                  (1,), mode=lax.GatherScatterMode.PROMISE_IN_BOUNDS)
              for k in range(FH // 16):
                ins[b][r0 + u, pl.ds(k * 16, 16)] = (
                    ins[b][r0 + u, pl.ds(k * 16, 16)] * aspl)
            return carry2
          lax.fori_loop(0, B // 16, rb, 0)

        def stage2(j, b, gnext):
          pltpu.make_async_copy(ff.at[bidxs[b]], ins[b], gsems[b]).wait()
          _scale(j, b)
          pltpu.sync_copy(ins[b], acc.at[dst_all.at[j]], add=True)
          if gnext:
            @pl.when(j + 3 < NBLK)
            def _():
              _bidx(j + 3, b)
              pltpu.async_copy(ff.at[bidxs[b]], ins[b], gsems[b])

        for b in range(3):
          _bidx(b, b)
          pltpu.async_copy(ff.at[bidxs[b]], ins[b], gsems[b])

        def p2(g, carry2):
          j = 3 * g
          stage2(j, 0, True)
          stage2(j + 1, 1, True)
          stage2(j + 2, 2, True)
          return carry2
        lax.fori_loop(0, NBLK // 3, p2, 0)
        stage2(NBLK - 1, 0, False)

        plsc.subcore_barrier()
        row0 = base + s * STRIPE
        pltpu.sync_copy(acc.at[pl.ds(s * STRIPE, STRIPE)],
                        agg.at[pl.ds(row0, STRIPE)])
        plsc.subcore_barrier()
        return carryh
      lax.fori_loop(0, FS, half_body, 0)
    return carry
  lax.fori_loop(0, 2, layer_body, 0)


def _sc_agg(srcs, dsts, ff, elcat, ercat):
  mesh = plsc.VectorSubcoreMesh(core_axis_name="c", subcore_axis_name="s")
  kfn = pl.kernel(
      _sc_body,
      out_type=jax.ShapeDtypeStruct((2 * H * FS * NP, FH), jnp.float32),
      mesh=mesh,
      compiler_params=pltpu.CompilerParams(needs_layout_passes=False,
                                           use_tc_tiling_on_sc=False),
      scratch_types=[
          pltpu.VMEM((NBLK, B), jnp.int32),
          pltpu.VMEM((NBLK, B), jnp.int32),
          pltpu.VMEM((NP,), jnp.float32),
          pltpu.VMEM((NP,), jnp.float32),
          pltpu.VMEM((NP,), jnp.float32),
          pltpu.VMEM((NBLK * B,), jnp.float32),
          pltpu.VMEM((B, FH), jnp.float32),
          pltpu.VMEM((B, FH), jnp.float32),
          pltpu.VMEM((B, FH), jnp.float32),
          pltpu.VMEM((B,), jnp.int32),
          pltpu.VMEM((B,), jnp.int32),
          pltpu.VMEM((B,), jnp.int32),
          pltpu.VMEM((B,), jnp.float32),
          pltpu.VMEM((B,), jnp.float32),
          pltpu.VMEM_SHARED((NP,), jnp.float32),
          pltpu.VMEM_SHARED((NP,), jnp.float32),
          pltpu.VMEM_SHARED((NP, FH), jnp.float32),
          pltpu.SemaphoreType.DMA,
          pltpu.SemaphoreType.DMA,
      ],
  )
  return kfn(srcs, dsts, ff, elcat, ercat)


def _out_body(agg_ref, wout_ref, bout_ref, o_ref):
  a = agg_ref[...]
  mls = []
  for l in range(2):
    parts = []
    for fh in range(FS):
      acc_fh = a[FS * (l * H + 0) + fh]
      for h in range(1, H):
        acc_fh = acc_fh + a[FS * (l * H + h) + fh]
      parts.append(acc_fh * 0.25)
    mls.append(jnp.concatenate(parts, axis=-1))
  h0 = jnp.maximum(mls[0], 0.0)
  h1 = jnp.maximum(h0 + mls[1], 0.0)
  lo = jnp.dot(h1, wout_ref[...], preferred_element_type=jnp.float32)
  lo = lo + bout_ref[...]
  z = lo - jnp.max(lo, axis=-1, keepdims=True)
  o_ref[...] = z - jnp.log(jnp.sum(jnp.exp(z), axis=-1, keepdims=True))


def _readout(agg, Wout, bout2):
  return pl.pallas_call(
      _out_body,
      grid=(NP // RB,),
      in_specs=[
          pl.BlockSpec((2 * H * FS, RB, FH), lambda i: (0, i, 0)),
          pl.BlockSpec((F, NCLASS), lambda i: (0, 0)),
          pl.BlockSpec((1, NCLASS), lambda i: (0, 0)),
      ],
      out_specs=pl.BlockSpec((RB, NCLASS), lambda i: (i, 0)),
      out_shape=jax.ShapeDtypeStruct((NP, NCLASS), jnp.float32),
  )(agg, Wout, bout2)


def kernel(x, edge_index, W0, al0, ar0, W1, al1, ar1, Wout, bout):
  src = edge_index[0].astype(jnp.int32)
  dst = edge_index[1].astype(jnp.int32)
  pad = EPAD - E
  srcs = jnp.concatenate([src, jnp.zeros((pad,), jnp.int32)])
  srcs = srcs.reshape(NTILE, NBLK, B)
  dsts = jnp.concatenate([dst, jnp.full((pad,), N, jnp.int32)])
  dsts = dsts.reshape(NTILE, NBLK, B)
  xp = jnp.pad(x, ((0, NP - N), (0, 0)))

  ff, el, er = _project(xp, W0, W1, al0, ar0, al1, ar1)
  er = er.at[:, N:].set(NEG)

  agg = _sc_agg(srcs, dsts, ff.reshape(2 * H * FS * NP, FH),
                el.reshape(2 * H * NP), er.reshape(2 * H * NP))

  out = _readout(agg.reshape(2 * H * FS, NP, FH), Wout,
                 bout.reshape(1, NCLASS).astype(jnp.float32))
  return out[:N]
